# baseline (speedup 1.0000x reference)
import functools

import jax
import jax.numpy as jnp
from jax import lax
from jax.experimental import pallas as pl
from jax.experimental.pallas import tpu as pltpu
from jax.experimental.pallas import tpu_sc as plsc

NC = 2
NS = 16
NW = NC * NS
CH = 128
BN = 1000
RB = 13


def _mesh():
    return plsc.VectorSubcoreMesh(core_axis_name="c", subcore_axis_name="s")


def _dg(v, idx):
    return lax.gather(
        v, idx[:, None],
        dimension_numbers=lax.GatherDimensionNumbers(
            offset_dims=(), collapsed_slice_dims=(0,),
            start_index_map=(0,)),
        slice_sizes=(1,),
        mode=lax.GatherScatterMode.PROMISE_IN_BOUNDS)


def _incl_prefix(mi, iota):
    x = mi
    for k in (1, 2, 4, 8):
        sh = _dg(x, jnp.maximum(iota - k, 0))
        x = x + jnp.where(iota >= k, sh, 0)
    return x


def _compact_perm(incl, iota):
    j = jnp.full((16,), -1, jnp.int32)
    tgt = iota + 1
    for step in (8, 4, 2, 1):
        tt = j + step
        vv = _dg(incl, tt)
        j = jnp.where(vv < tgt, tt, j)
    return jnp.minimum(j + 1, 15)


def _make_deg(C, D1, R1):
    @functools.partial(
        pl.kernel,
        out_type=[
            jax.ShapeDtypeStruct((D1,), jnp.float32),
            jax.ShapeDtypeStruct((D1,), jnp.float32),
        ],
        mesh=_mesh(),
        scratch_types=[
            pltpu.VMEM((C, CH), jnp.int32),
            pltpu.VMEM((CH,), jnp.float32),
            pltpu.VMEM((R1,), jnp.float32),
            pltpu.VMEM_SHARED((D1,), jnp.float32),
        ],
    )
    def deg(dsts_hbm, zeros_hbm, out0_hbm, out1_hbm, dst_v, ones_v, zv, acc):
        c = lax.axis_index("c")
        s = lax.axis_index("s")
        wid = c * NS + s
        pltpu.sync_copy(dsts_hbm.at[pl.ds(wid * C, C)], dst_v)
        pltpu.sync_copy(zeros_hbm, zv)
        pltpu.sync_copy(zv, acc.at[pl.ds(s * R1, R1)])
        for i in range(CH // 16):
            ones_v[pl.ds(i * 16, 16)] = jnp.ones((16,), jnp.float32)
        plsc.subcore_barrier()

        def body(j, carry):
            pltpu.sync_copy(ones_v, acc.at[dst_v.at[j]], add=True)
            return carry

        lax.fori_loop(0, C, body, 0)
        plsc.subcore_barrier()

        pltpu.sync_copy(acc.at[pl.ds(s * R1, R1)], zv)

        @pl.when(c == 0)
        def _():
            pltpu.sync_copy(zv, out0_hbm.at[pl.ds(s * R1, R1)])

        @pl.when(c == 1)
        def _():
            pltpu.sync_copy(zv, out1_hbm.at[pl.ds(s * R1, R1)])

    return deg


def _make_part(C, CP, LSZ, H, TRASH):
    @functools.partial(
        pl.kernel,
        out_type=[
            jax.ShapeDtypeStruct((2 * NW * LSZ,), jnp.int32),
            jax.ShapeDtypeStruct((2 * NW * 8,), jnp.int32),
        ],
        mesh=_mesh(),
        scratch_types=[
            pltpu.VMEM((C, CH), jnp.int32),
            pltpu.VMEM((C, CH), jnp.int32),
            pltpu.VMEM((LSZ + 16,), jnp.int32),
            pltpu.VMEM((LSZ + 16,), jnp.int32),
            pltpu.VMEM((16,), jnp.int32),
        ],
    )
    def part(srcs_hbm, dsts_hbm, plist_hbm, cnt_hbm,
             src_v, dst_v, ol0, ol1, cnt_v):
        c = lax.axis_index("c")
        s = lax.axis_index("s")
        wid = c * NS + s
        pltpu.sync_copy(srcs_hbm.at[pl.ds(wid * C, C)], src_v)
        pltpu.sync_copy(dsts_hbm.at[pl.ds(wid * C, C)], dst_v)

        iota = lax.iota(jnp.int32, 16)

        def chunk_body(j, ps):
            p0, p1 = ps
            for l in range(CH // 16):
                sl = pl.ds(l * 16, 16)
                d = dst_v[j, sl]
                sv = src_v[j, sl]
                m0i = jnp.where(d < H, 1, 0)
                incl0 = _incl_prefix(m0i, iota)
                perm0 = _compact_perm(incl0, iota)
                vals0 = _dg((sv << RB) | d, perm0)
                ol0[pl.ds(p0, 16)] = vals0
                p0 = p0 + incl0[15]
                m1i = 1 - m0i
                incl1 = _incl_prefix(m1i, iota)
                perm1 = _compact_perm(incl1, iota)
                spread = H + ((j * 16 + iota) & 63)
                rv1 = jnp.where(d - H < H, d - H, spread)
                vals1 = _dg((sv << RB) | rv1, perm1)
                ol1[pl.ds(p1, 16)] = vals1
                p1 = p1 + incl1[15]
            return (p0, p1)

        p0, p1 = lax.fori_loop(0, C, chunk_body,
                               (jnp.int32(0), jnp.int32(0)))

        for k in range(CH // 16):
            ts = H + ((k * 16 + iota) & 63)
            ol0[pl.ds(p0 + k * 16, 16)] = ts
            ol1[pl.ds(p1 + k * 16, 16)] = ts

        n0 = lax.shift_right_logical(p0 + (CH - 1), 7)
        n1 = lax.shift_right_logical(p1 + (CH - 1), 7)
        cnt_v[...] = (jnp.where(iota == 0, n0, 0)
                      + jnp.where(iota == 8, n1, 0))

        pltpu.sync_copy(ol0.at[pl.ds(0, LSZ)],
                        plist_hbm.at[pl.ds(wid * LSZ, LSZ)])
        pltpu.sync_copy(ol1.at[pl.ds(0, LSZ)],
                        plist_hbm.at[pl.ds((NW + wid) * LSZ, LSZ)])
        pltpu.sync_copy(cnt_v.at[pl.ds(0, 8)],
                        cnt_hbm.at[pl.ds(wid * 8, 8)])
        pltpu.sync_copy(cnt_v.at[pl.ds(8, 8)],
                        cnt_hbm.at[pl.ds((NW + wid) * 8, 8)])

    return part


def _make_agg(D, CP, LSZ, AR, RPT):
    part = jax.ShapeDtypeStruct((AR, D), jnp.float32)

    @functools.partial(
        pl.kernel,
        out_type=[part, part],
        mesh=_mesh(),
        scratch_types=[
            pltpu.VMEM((2 * LSZ,), jnp.int32),
            pltpu.VMEM((2, CH), jnp.int32),
            pltpu.VMEM((2, CH), jnp.int32),
            pltpu.VMEM((16,), jnp.int32),
            pltpu.VMEM((CH, D), jnp.float32),
            pltpu.VMEM((CH, D), jnp.float32),
            pltpu.VMEM((CH, D), jnp.float32),
            pltpu.VMEM_SHARED((AR, D), jnp.float32),
            pltpu.SemaphoreType.DMA,
            pltpu.SemaphoreType.DMA,
        ],
    )
    def agg(g_hbm, plist_hbm, cnt_hbm, zrows_hbm, out0_hbm, out1_hbm,
            plist_v, sidx2, idx2, cnt_v, buf0, buf1, zbuf, acc, sem0, sem1):
        c = lax.axis_index("c")
        s = lax.axis_index("s")
        base = (c * NW + 2 * s) * LSZ
        pltpu.sync_copy(plist_hbm.at[pl.ds(base, 2 * LSZ)], plist_v)
        pltpu.sync_copy(cnt_hbm.at[pl.ds((c * NW + 2 * s) * 8, 16)], cnt_v)
        pltpu.sync_copy(zrows_hbm, zbuf)

        cv = cnt_v[...]
        nA = cv[0]
        nB = cv[8]
        n = nA + nB

        off = 0
        while off < RPT:
            m = min(CH, RPT - off)
            pltpu.sync_copy(zbuf.at[pl.ds(0, m)],
                            acc.at[pl.ds(s * RPT + off, m)])
            off += m
        plsc.subcore_barrier()

        def issue(j, pb, buf, sem):
            o = jnp.where(j < nA, j, CP + (j - nA)) * CH
            for l in range(CH // 16):
                w = plist_v[pl.ds(o + l * 16, 16)]
                sidx2[pb, pl.ds(l * 16, 16)] = lax.shift_right_logical(w, RB)
                idx2[pb, pl.ds(l * 16, 16)] = w & ((1 << RB) - 1)
            pltpu.async_copy(g_hbm.at[sidx2.at[pb]], buf, sem)

        @pl.when(n > 0)
        def _():
            issue(0, 0, buf0, sem0)

        @pl.when(n > 1)
        def _():
            issue(1, 1, buf1, sem1)

        def body(i, carry):
            j = 2 * i
            pltpu.make_async_copy(g_hbm.at[sidx2.at[0]], buf0, sem0).wait()
            pltpu.sync_copy(buf0, acc.at[idx2.at[0]], add=True)

            @pl.when(j + 2 < n)
            def _():
                issue(j + 2, 0, buf0, sem0)

            @pl.when(j + 1 < n)
            def _():
                pltpu.make_async_copy(g_hbm.at[sidx2.at[1]],
                                      buf1, sem1).wait()
                pltpu.sync_copy(buf1, acc.at[idx2.at[1]], add=True)

                @pl.when(j + 3 < n)
                def _():
                    issue(j + 3, 1, buf1, sem1)

            return carry

        lax.fori_loop(0, lax.shift_right_logical(n + 1, 1), body, 0)
        plsc.subcore_barrier()

        off = 0
        while off < RPT:
            m = min(CH, RPT - off)
            pltpu.sync_copy(acc.at[pl.ds(s * RPT + off, m)],
                            buf0.at[pl.ds(0, m)])

            @pl.when(c == 0)
            def _():
                pltpu.sync_copy(buf0.at[pl.ds(0, m)],
                                out0_hbm.at[pl.ds(s * RPT + off, m)])

            @pl.when(c == 1)
            def _():
                pltpu.sync_copy(buf0.at[pl.ds(0, m)],
                                out1_hbm.at[pl.ds(s * RPT + off, m)])

            off += m

    return agg


def _tc_first(d0, d1, x, W):
    N, D = x.shape

    def body(d0_ref, d1_ref, x_ref, w_ref, o_ref):
        dinv = lax.rsqrt(d0_ref[...] + d1_ref[...] + 1.0)
        h = jnp.dot(x_ref[...], w_ref[...], preferred_element_type=jnp.float32)
        o_ref[...] = dinv * h

    return pl.pallas_call(
        body,
        grid=(N // BN,),
        in_specs=[
            pl.BlockSpec((BN, 1), lambda i: (i, 0)),
            pl.BlockSpec((BN, 1), lambda i: (i, 0)),
            pl.BlockSpec((BN, D), lambda i: (i, 0)),
            pl.BlockSpec((D, D), lambda i: (0, 0)),
        ],
        out_specs=pl.BlockSpec((BN, D), lambda i: (i, 0)),
        out_shape=jax.ShapeDtypeStruct((N, D), jnp.float32),
    )(d0, d1, x, W)


def _part_specs(D, PB):
    return [
        pl.BlockSpec((BN, D), lambda i: (jnp.minimum(i, PB - 1), 0)),
        pl.BlockSpec((BN, D), lambda i: (jnp.maximum(i - PB, 0), 0)),
    ]


def _tc_mid(d0, d1, parts, g, b, W, PB):
    N, D = g.shape

    def body(d0_ref, d1_ref, p0, p1, g_ref, b_ref, w_ref, o_ref):
        i = pl.program_id(0)
        dinv = lax.rsqrt(d0_ref[...] + d1_ref[...] + 1.0)
        p = jnp.where(i < PB, p0[...], p1[...])
        xl = dinv * (p + g_ref[...]) + b_ref[...]
        xl = jnp.maximum(xl, 0.0)
        h = jnp.dot(xl, w_ref[...], preferred_element_type=jnp.float32)
        o_ref[...] = dinv * h

    return pl.pallas_call(
        body,
        grid=(N // BN,),
        in_specs=[
            pl.BlockSpec((BN, 1), lambda i: (i, 0)),
            pl.BlockSpec((BN, 1), lambda i: (i, 0)),
            *_part_specs(D, PB),
            pl.BlockSpec((BN, D), lambda i: (i, 0)),
            pl.BlockSpec((1, D), lambda i: (0, 0)),
            pl.BlockSpec((D, D), lambda i: (0, 0)),
        ],
        out_specs=pl.BlockSpec((BN, D), lambda i: (i, 0)),
        out_shape=jax.ShapeDtypeStruct((N, D), jnp.float32),
    )(d0, d1, *parts, g, b, W)


def _tc_last(d0, d1, parts, g, b, PB):
    N, D = g.shape

    def body(d0_ref, d1_ref, p0, p1, g_ref, b_ref, o_ref):
        i = pl.program_id(0)
        dinv = lax.rsqrt(d0_ref[...] + d1_ref[...] + 1.0)
        p = jnp.where(i < PB, p0[...], p1[...])
        o_ref[...] = dinv * (p + g_ref[...]) + b_ref[...]

    return pl.pallas_call(
        body,
        grid=(N // BN,),
        in_specs=[
            pl.BlockSpec((BN, 1), lambda i: (i, 0)),
            pl.BlockSpec((BN, 1), lambda i: (i, 0)),
            *_part_specs(D, PB),
            pl.BlockSpec((BN, D), lambda i: (i, 0)),
            pl.BlockSpec((1, D), lambda i: (0, 0)),
        ],
        out_specs=pl.BlockSpec((BN, D), lambda i: (i, 0)),
        out_shape=jax.ShapeDtypeStruct((N, D), jnp.float32),
    )(d0, d1, *parts, g, b)


def kernel(x, edge_index, W1, b1, W2, b2, W3, b3):
    N, D = x.shape
    E = edge_index.shape[1]
    assert N % (2 * BN) == 0 and D == 128

    C = -(-E // (NW * CH))
    C += C % 2
    E_pad = NW * C * CH
    CP = C + 1
    LSZ = CP * CH

    R1 = -(-(N + 1) // NS)
    R1 += (-R1) % 8
    D1 = R1 * NS

    H = N // 2
    PB = H // BN
    RPT = -(-(H + 2) // NS)
    RPT += (-RPT) % 8
    AR = RPT * NS
    TRASH = AR - 1
    assert N < (1 << (31 - RB)) and TRASH < (1 << RB)

    src = edge_index[0]
    dst = edge_index[1]
    pad = E_pad - E
    srcs = jnp.concatenate([src, jnp.zeros((pad,), src.dtype)]).reshape(NW * C, CH)
    dsts = jnp.concatenate([dst, jnp.full((pad,), N, dst.dtype)]).reshape(NW * C, CH)

    zeros1 = jnp.zeros((R1,), jnp.float32)
    zrows = jnp.zeros((CH, D), jnp.float32)

    deg0, deg1 = _make_deg(C, D1, R1)(dsts, zeros1)
    d0 = deg0[:N].reshape(N, 1)
    d1 = deg1[:N].reshape(N, 1)

    plist, cnts = _make_part(C, CP, LSZ, H, TRASH)(srcs, dsts)

    agg = _make_agg(D, CP, LSZ, AR, RPT)
    b1r, b2r, b3r = (b.reshape(1, D) for b in (b1, b2, b3))

    g1 = _tc_first(d0, d1, x, W1)
    parts = agg(g1, plist, cnts, zrows)
    g2 = _tc_mid(d0, d1, parts, g1, b1r, W2, PB)
    parts = agg(g2, plist, cnts, zrows)
    g3 = _tc_mid(d0, d1, parts, g2, b2r, W3, PB)
    parts = agg(g3, plist, cnts, zrows)
    return _tc_last(d0, d1, parts, g3, b3r, PB)


# speedup vs baseline: 1.1734x; 1.1734x over previous
import functools

import jax
import jax.numpy as jnp
from jax import lax
from jax.experimental import pallas as pl
from jax.experimental.pallas import tpu as pltpu
from jax.experimental.pallas import tpu_sc as plsc

NC = 2
NS = 16
NW = NC * NS
CH = 128
BN = 1000
RB = 13


def _mesh():
    return plsc.VectorSubcoreMesh(core_axis_name="c", subcore_axis_name="s")


def _dg(v, idx):
    return lax.gather(
        v, idx[:, None],
        dimension_numbers=lax.GatherDimensionNumbers(
            offset_dims=(), collapsed_slice_dims=(0,),
            start_index_map=(0,)),
        slice_sizes=(1,),
        mode=lax.GatherScatterMode.PROMISE_IN_BOUNDS)


def _incl_prefix(mi, iota):
    x = mi
    for k in (1, 2, 4, 8):
        sh = _dg(x, jnp.maximum(iota - k, 0))
        x = x + jnp.where(iota >= k, sh, 0)
    return x


def _compact_perm(incl, iota):
    j = jnp.full((16,), -1, jnp.int32)
    tgt = iota + 1
    for step in (8, 4, 2, 1):
        tt = j + step
        vv = _dg(incl, tt)
        j = jnp.where(vv < tgt, tt, j)
    return jnp.minimum(j + 1, 15)


def _make_deg(C, D1, R1):
    @functools.partial(
        pl.kernel,
        out_type=[
            jax.ShapeDtypeStruct((D1,), jnp.float32),
            jax.ShapeDtypeStruct((D1,), jnp.float32),
        ],
        mesh=_mesh(),
        scratch_types=[
            pltpu.VMEM((C, CH), jnp.int32),
            pltpu.VMEM((CH,), jnp.float32),
            pltpu.VMEM((R1,), jnp.float32),
            pltpu.VMEM_SHARED((D1,), jnp.float32),
        ],
    )
    def deg(dsts_hbm, zeros_hbm, out0_hbm, out1_hbm, dst_v, ones_v, zv, acc):
        c = lax.axis_index("c")
        s = lax.axis_index("s")
        wid = c * NS + s
        pltpu.sync_copy(dsts_hbm.at[pl.ds(wid * C, C)], dst_v)
        pltpu.sync_copy(zeros_hbm, zv)
        pltpu.sync_copy(zv, acc.at[pl.ds(s * R1, R1)])
        for i in range(CH // 16):
            ones_v[pl.ds(i * 16, 16)] = jnp.ones((16,), jnp.float32)
        plsc.subcore_barrier()

        def body(j, carry):
            pltpu.sync_copy(ones_v, acc.at[dst_v.at[j]], add=True)
            return carry

        lax.fori_loop(0, C, body, 0)
        plsc.subcore_barrier()

        pltpu.sync_copy(acc.at[pl.ds(s * R1, R1)], zv)

        @pl.when(c == 0)
        def _():
            pltpu.sync_copy(zv, out0_hbm.at[pl.ds(s * R1, R1)])

        @pl.when(c == 1)
        def _():
            pltpu.sync_copy(zv, out1_hbm.at[pl.ds(s * R1, R1)])

    return deg


def _make_part(C, CP, LSZ, H, TRASH):
    @functools.partial(
        pl.kernel,
        out_type=[
            jax.ShapeDtypeStruct((2 * NW * LSZ,), jnp.int32),
            jax.ShapeDtypeStruct((2 * NW * 8,), jnp.int32),
        ],
        mesh=_mesh(),
        scratch_types=[
            pltpu.VMEM((C, CH), jnp.int32),
            pltpu.VMEM((C, CH), jnp.int32),
            pltpu.VMEM((LSZ + 16,), jnp.int32),
            pltpu.VMEM((LSZ + 16,), jnp.int32),
            pltpu.VMEM((16,), jnp.int32),
        ],
    )
    def part(srcs_hbm, dsts_hbm, plist_hbm, cnt_hbm,
             src_v, dst_v, ol0, ol1, cnt_v):
        c = lax.axis_index("c")
        s = lax.axis_index("s")
        wid = c * NS + s
        pltpu.sync_copy(srcs_hbm.at[pl.ds(wid * C, C)], src_v)
        pltpu.sync_copy(dsts_hbm.at[pl.ds(wid * C, C)], dst_v)

        iota = lax.iota(jnp.int32, 16)

        def chunk_body(j, ps):
            p0, p1 = ps
            for l in range(CH // 16):
                sl = pl.ds(l * 16, 16)
                d = dst_v[j, sl]
                sv = src_v[j, sl]
                m0i = jnp.where(d < H, 1, 0)
                incl0 = _incl_prefix(m0i, iota)
                perm0 = _compact_perm(incl0, iota)
                vals0 = _dg((sv << RB) | d, perm0)
                ol0[pl.ds(p0, 16)] = vals0
                p0 = p0 + incl0[15]
                m1i = 1 - m0i
                incl1 = _incl_prefix(m1i, iota)
                perm1 = _compact_perm(incl1, iota)
                spread = H + ((j * 16 + iota) & 63)
                rv1 = jnp.where(d - H < H, d - H, spread)
                vals1 = _dg((sv << RB) | rv1, perm1)
                ol1[pl.ds(p1, 16)] = vals1
                p1 = p1 + incl1[15]
            return (p0, p1)

        p0, p1 = lax.fori_loop(0, C, chunk_body,
                               (jnp.int32(0), jnp.int32(0)))

        for k in range(CH // 16):
            ts = H + ((k * 16 + iota) & 63)
            ol0[pl.ds(p0 + k * 16, 16)] = ts
            ol1[pl.ds(p1 + k * 16, 16)] = ts

        n0 = lax.shift_right_logical(p0 + (CH - 1), 7)
        n1 = lax.shift_right_logical(p1 + (CH - 1), 7)
        cnt_v[...] = (jnp.where(iota == 0, n0, 0)
                      + jnp.where(iota == 8, n1, 0))

        pltpu.sync_copy(ol0.at[pl.ds(0, LSZ)],
                        plist_hbm.at[pl.ds(wid * LSZ, LSZ)])
        pltpu.sync_copy(ol1.at[pl.ds(0, LSZ)],
                        plist_hbm.at[pl.ds((NW + wid) * LSZ, LSZ)])
        pltpu.sync_copy(cnt_v.at[pl.ds(0, 8)],
                        cnt_hbm.at[pl.ds(wid * 8, 8)])
        pltpu.sync_copy(cnt_v.at[pl.ds(8, 8)],
                        cnt_hbm.at[pl.ds((NW + wid) * 8, 8)])

    return part


def _make_agg(D, CP, LSZ, AR, RPT):
    part = jax.ShapeDtypeStruct((AR, D), jnp.float32)

    @functools.partial(
        pl.kernel,
        out_type=[part, part],
        mesh=_mesh(),
        scratch_types=[
            pltpu.VMEM((2 * LSZ,), jnp.int32),
            pltpu.VMEM((2, CH), jnp.int32),
            pltpu.VMEM((2, CH), jnp.int32),
            pltpu.VMEM((16,), jnp.int32),
            pltpu.VMEM((CH, D), jnp.float32),
            pltpu.VMEM((CH, D), jnp.float32),
            pltpu.VMEM((CH, D), jnp.float32),
            pltpu.VMEM_SHARED((AR, D), jnp.float32),
            pltpu.SemaphoreType.DMA,
            pltpu.SemaphoreType.DMA,
        ],
    )
    def agg(g_hbm, plist_hbm, cnt_hbm, zrows_hbm, out0_hbm, out1_hbm,
            plist_v, sidx2, idx2, cnt_v, buf0, buf1, zbuf, acc, sem0, sem1):
        c = lax.axis_index("c")
        s = lax.axis_index("s")
        base = (c * NW + 2 * s) * LSZ
        pltpu.sync_copy(plist_hbm.at[pl.ds(base, 2 * LSZ)], plist_v)
        pltpu.sync_copy(cnt_hbm.at[pl.ds((c * NW + 2 * s) * 8, 16)], cnt_v)
        pltpu.sync_copy(zrows_hbm, zbuf)

        cv = cnt_v[...]
        nA = cv[0]
        nB = cv[8]
        n = nA + nB

        off = 0
        while off < RPT:
            m = min(CH, RPT - off)
            pltpu.sync_copy(zbuf.at[pl.ds(0, m)],
                            acc.at[pl.ds(s * RPT + off, m)])
            off += m
        plsc.subcore_barrier()

        def issue(j, pb, buf, sem):
            o = jnp.where(j < nA, j, CP + (j - nA)) * CH
            for l in range(CH // 16):
                w = plist_v[pl.ds(o + l * 16, 16)]
                sidx2[pb, pl.ds(l * 16, 16)] = lax.shift_right_logical(w, RB)
                idx2[pb, pl.ds(l * 16, 16)] = w & ((1 << RB) - 1)
            pltpu.async_copy(g_hbm.at[sidx2.at[pb]], buf, sem)

        @pl.when(n > 0)
        def _():
            issue(0, 0, buf0, sem0)

        @pl.when(n > 1)
        def _():
            issue(1, 1, buf1, sem1)

        def body(i, carry):
            j = 2 * i
            pltpu.make_async_copy(g_hbm.at[sidx2.at[0]], buf0, sem0).wait()
            pltpu.sync_copy(buf0, acc.at[idx2.at[0]], add=True)

            @pl.when(j + 2 < n)
            def _():
                issue(j + 2, 0, buf0, sem0)

            @pl.when(j + 1 < n)
            def _():
                pltpu.make_async_copy(g_hbm.at[sidx2.at[1]],
                                      buf1, sem1).wait()
                pltpu.sync_copy(buf1, acc.at[idx2.at[1]], add=True)

                @pl.when(j + 3 < n)
                def _():
                    issue(j + 3, 1, buf1, sem1)

            return carry

        lax.fori_loop(0, lax.shift_right_logical(n + 1, 1), body, 0)
        plsc.subcore_barrier()

        off = 0
        while off < RPT:
            m = min(CH, RPT - off)
            pltpu.sync_copy(acc.at[pl.ds(s * RPT + off, m)],
                            buf0.at[pl.ds(0, m)])

            @pl.when(c == 0)
            def _():
                pltpu.sync_copy(buf0.at[pl.ds(0, m)],
                                out0_hbm.at[pl.ds(s * RPT + off, m)])

            @pl.when(c == 1)
            def _():
                pltpu.sync_copy(buf0.at[pl.ds(0, m)],
                                out1_hbm.at[pl.ds(s * RPT + off, m)])

            off += m

    return agg


def _tc_first(d0, d1, x, W):
    N, D = x.shape

    def body(d0_ref, d1_ref, x_ref, w_ref, o_ref):
        dinv = lax.rsqrt(d0_ref[...] + d1_ref[...] + 1.0)
        h = jnp.dot(x_ref[...], w_ref[...], preferred_element_type=jnp.float32)
        o_ref[...] = dinv * h

    return pl.pallas_call(
        body,
        grid=(N // BN,),
        in_specs=[
            pl.BlockSpec((BN, 1), lambda i: (i, 0)),
            pl.BlockSpec((BN, 1), lambda i: (i, 0)),
            pl.BlockSpec((BN, D), lambda i: (i, 0)),
            pl.BlockSpec((D, D), lambda i: (0, 0)),
        ],
        out_specs=pl.BlockSpec((BN, D), lambda i: (i, 0)),
        out_shape=jax.ShapeDtypeStruct((N, D), jnp.float32),
    )(d0, d1, x, W)


def _part_specs(D, PB):
    return [
        pl.BlockSpec((BN, D), lambda i: (jnp.minimum(i, PB - 1), 0)),
        pl.BlockSpec((BN, D), lambda i: (jnp.maximum(i - PB, 0), 0)),
    ]


def _tc_mid(d0, d1, parts, g, b, W, PB):
    N, D = g.shape

    def body(d0_ref, d1_ref, p0, p1, g_ref, b_ref, w_ref, o_ref):
        i = pl.program_id(0)
        dinv = lax.rsqrt(d0_ref[...] + d1_ref[...] + 1.0)
        p = jnp.where(i < PB, p0[...], p1[...])
        xl = dinv * (p + g_ref[...]) + b_ref[...]
        xl = jnp.maximum(xl, 0.0)
        h = jnp.dot(xl, w_ref[...], preferred_element_type=jnp.float32)
        o_ref[...] = dinv * h

    return pl.pallas_call(
        body,
        grid=(N // BN,),
        in_specs=[
            pl.BlockSpec((BN, 1), lambda i: (i, 0)),
            pl.BlockSpec((BN, 1), lambda i: (i, 0)),
            *_part_specs(D, PB),
            pl.BlockSpec((BN, D), lambda i: (i, 0)),
            pl.BlockSpec((1, D), lambda i: (0, 0)),
            pl.BlockSpec((D, D), lambda i: (0, 0)),
        ],
        out_specs=pl.BlockSpec((BN, D), lambda i: (i, 0)),
        out_shape=jax.ShapeDtypeStruct((N, D), jnp.float32),
    )(d0, d1, *parts, g, b, W)


def _tc_last(d0, d1, parts, g, b, PB):
    N, D = g.shape

    def body(d0_ref, d1_ref, p0, p1, g_ref, b_ref, o_ref):
        i = pl.program_id(0)
        dinv = lax.rsqrt(d0_ref[...] + d1_ref[...] + 1.0)
        p = jnp.where(i < PB, p0[...], p1[...])
        o_ref[...] = dinv * (p + g_ref[...]) + b_ref[...]

    return pl.pallas_call(
        body,
        grid=(N // BN,),
        in_specs=[
            pl.BlockSpec((BN, 1), lambda i: (i, 0)),
            pl.BlockSpec((BN, 1), lambda i: (i, 0)),
            *_part_specs(D, PB),
            pl.BlockSpec((BN, D), lambda i: (i, 0)),
            pl.BlockSpec((1, D), lambda i: (0, 0)),
        ],
        out_specs=pl.BlockSpec((BN, D), lambda i: (i, 0)),
        out_shape=jax.ShapeDtypeStruct((N, D), jnp.float32),
    )(d0, d1, *parts, g, b)


def kernel(x, edge_index, W1, b1, W2, b2, W3, b3):
    N, D = x.shape
    E = edge_index.shape[1]
    assert N % (2 * BN) == 0 and D == 128

    C = -(-E // (NW * CH))
    C += C % 2
    E_pad = NW * C * CH
    CP = C + 1
    LSZ = CP * CH

    R1 = -(-(N + 1) // NS)
    R1 += (-R1) % 8
    D1 = R1 * NS

    H = N // 2
    PB = H // BN
    RPT = -(-(H + 2) // NS)
    RPT += (-RPT) % 8
    AR = RPT * NS
    TRASH = AR - 1
    assert N < (1 << (31 - RB)) and TRASH < (1 << RB)

    src = edge_index[0]
    dst = edge_index[1]
    pad = E_pad - E

    def _chunks(a):
        return a.reshape(C, NW, CH).transpose(1, 0, 2).reshape(NW * C, CH)

    srcs = _chunks(jnp.concatenate([src, jnp.zeros((pad,), src.dtype)]))
    dsts = _chunks(jnp.concatenate([dst, jnp.full((pad,), N, dst.dtype)]))

    zeros1 = jnp.zeros((R1,), jnp.float32)
    zrows = jnp.zeros((CH, D), jnp.float32)

    deg0, deg1 = _make_deg(C, D1, R1)(dsts, zeros1)
    d0 = deg0[:N].reshape(N, 1)
    d1 = deg1[:N].reshape(N, 1)

    plist, cnts = _make_part(C, CP, LSZ, H, TRASH)(srcs, dsts)

    agg = _make_agg(D, CP, LSZ, AR, RPT)
    b1r, b2r, b3r = (b.reshape(1, D) for b in (b1, b2, b3))

    g1 = _tc_first(d0, d1, x, W1)
    parts = agg(g1, plist, cnts, zrows)
    g2 = _tc_mid(d0, d1, parts, g1, b1r, W2, PB)
    parts = agg(g2, plist, cnts, zrows)
    g3 = _tc_mid(d0, d1, parts, g2, b2r, W3, PB)
    parts = agg(g3, plist, cnts, zrows)
    return _tc_last(d0, d1, parts, g3, b3r, PB)


# speedup vs baseline: 2.3855x; 2.0329x over previous
import functools

import jax
import jax.numpy as jnp
from jax import lax
from jax.experimental import pallas as pl
from jax.experimental.pallas import tpu as pltpu
from jax.experimental.pallas import tpu_sc as plsc

NC = 2
NS = 16
NW = NC * NS
CH = 128
BN = 1000
RB = 13


def _mesh():
    return plsc.VectorSubcoreMesh(core_axis_name="c", subcore_axis_name="s")


def _dg(v, idx):
    return lax.gather(
        v, idx[:, None],
        dimension_numbers=lax.GatherDimensionNumbers(
            offset_dims=(), collapsed_slice_dims=(0,),
            start_index_map=(0,)),
        slice_sizes=(1,),
        mode=lax.GatherScatterMode.PROMISE_IN_BOUNDS)


def _incl_prefix(mi, iota):
    x = mi
    for k in (1, 2, 4, 8):
        sh = _dg(x, jnp.maximum(iota - k, 0))
        x = x + jnp.where(iota >= k, sh, 0)
    return x


def _compact_perm(incl, iota):
    j = jnp.full((16,), -1, jnp.int32)
    tgt = iota + 1
    for step in (8, 4, 2, 1):
        tt = j + step
        vv = _dg(incl, tt)
        j = jnp.where(vv < tgt, tt, j)
    return jnp.minimum(j + 1, 15)


def _make_deg(C, D1, R1):
    @functools.partial(
        pl.kernel,
        out_type=[
            jax.ShapeDtypeStruct((D1,), jnp.float32),
            jax.ShapeDtypeStruct((D1,), jnp.float32),
        ],
        mesh=_mesh(),
        scratch_types=[
            pltpu.VMEM((C, CH), jnp.int32),
            pltpu.VMEM((CH,), jnp.float32),
            pltpu.VMEM((R1,), jnp.float32),
            pltpu.VMEM_SHARED((D1,), jnp.float32),
        ],
    )
    def deg(dsts_hbm, zeros_hbm, out0_hbm, out1_hbm, dst_v, ones_v, zv, acc):
        c = lax.axis_index("c")
        s = lax.axis_index("s")
        wid = c * NS + s
        pltpu.sync_copy(dsts_hbm.at[pl.ds(wid * C, C)], dst_v)
        pltpu.sync_copy(zeros_hbm, zv)
        pltpu.sync_copy(zv, acc.at[pl.ds(s * R1, R1)])
        for i in range(CH // 16):
            ones_v[pl.ds(i * 16, 16)] = jnp.ones((16,), jnp.float32)
        plsc.subcore_barrier()

        def body(j, carry):
            pltpu.sync_copy(ones_v, acc.at[dst_v.at[j]], add=True)
            return carry

        lax.fori_loop(0, C, body, 0)
        plsc.subcore_barrier()

        pltpu.sync_copy(acc.at[pl.ds(s * R1, R1)], zv)

        @pl.when(c == 0)
        def _():
            pltpu.sync_copy(zv, out0_hbm.at[pl.ds(s * R1, R1)])

        @pl.when(c == 1)
        def _():
            pltpu.sync_copy(zv, out1_hbm.at[pl.ds(s * R1, R1)])

    return deg


def _make_part(C, CP, LSZ, H, TRASH):
    @functools.partial(
        pl.kernel,
        out_type=[
            jax.ShapeDtypeStruct((2 * NW * LSZ,), jnp.int32),
            jax.ShapeDtypeStruct((2 * NW * 8,), jnp.int32),
        ],
        mesh=_mesh(),
        scratch_types=[
            pltpu.VMEM((C, CH), jnp.int32),
            pltpu.VMEM((C, CH), jnp.int32),
            pltpu.VMEM((LSZ + 16,), jnp.int32),
            pltpu.VMEM((LSZ + 16,), jnp.int32),
            pltpu.VMEM((16,), jnp.int32),
        ],
    )
    def part(srcs_hbm, dsts_hbm, plist_hbm, cnt_hbm,
             src_v, dst_v, ol0, ol1, cnt_v):
        c = lax.axis_index("c")
        s = lax.axis_index("s")
        wid = c * NS + s
        pltpu.sync_copy(srcs_hbm.at[pl.ds(wid * C, C)], src_v)
        pltpu.sync_copy(dsts_hbm.at[pl.ds(wid * C, C)], dst_v)

        iota = lax.iota(jnp.int32, 16)

        def chunk_body(j, ps):
            p0, p1 = ps
            for l in range(CH // 16):
                sl = pl.ds(l * 16, 16)
                d = dst_v[j, sl]
                sv = src_v[j, sl]
                m0i = jnp.where(d < H, 1, 0)
                incl0 = _incl_prefix(m0i, iota)
                perm0 = _compact_perm(incl0, iota)
                vals0 = _dg((sv << RB) | d, perm0)
                ol0[pl.ds(p0, 16)] = vals0
                p0 = p0 + incl0[15]
                m1i = (1 - m0i) * jnp.where(d - H < H, 1, 0)
                incl1 = _incl_prefix(m1i, iota)
                perm1 = _compact_perm(incl1, iota)
                spread = H + ((j * 16 + iota) & 63)
                rv1 = jnp.where(d - H < H, d - H, spread)
                vals1 = _dg((sv << RB) | rv1, perm1)
                ol1[pl.ds(p1, 16)] = vals1
                p1 = p1 + incl1[15]
            return (p0, p1)

        p0, p1 = lax.fori_loop(0, C, chunk_body,
                               (jnp.int32(0), jnp.int32(0)))

        for k in range(CH // 16):
            ts = H + ((k * 16 + iota) & 63)
            ol0[pl.ds(p0 + k * 16, 16)] = ts
            ol1[pl.ds(p1 + k * 16, 16)] = ts

        n0 = lax.shift_right_logical(p0 + (CH - 1), 7)
        n1 = lax.shift_right_logical(p1 + (CH - 1), 7)
        cnt_v[...] = (jnp.where(iota == 0, n0, 0)
                      + jnp.where(iota == 8, n1, 0))

        pltpu.sync_copy(ol0.at[pl.ds(0, LSZ)],
                        plist_hbm.at[pl.ds(wid * LSZ, LSZ)])
        pltpu.sync_copy(ol1.at[pl.ds(0, LSZ)],
                        plist_hbm.at[pl.ds((NW + wid) * LSZ, LSZ)])
        pltpu.sync_copy(cnt_v.at[pl.ds(0, 8)],
                        cnt_hbm.at[pl.ds(wid * 8, 8)])
        pltpu.sync_copy(cnt_v.at[pl.ds(8, 8)],
                        cnt_hbm.at[pl.ds((NW + wid) * 8, 8)])

    return part


def _make_agg(D, CP, LSZ, AR, RPT):
    part = jax.ShapeDtypeStruct((AR, D), jnp.float32)

    @functools.partial(
        pl.kernel,
        out_type=[part, part],
        mesh=_mesh(),
        scratch_types=[
            pltpu.VMEM((2 * LSZ,), jnp.int32),
            pltpu.VMEM((2, CH), jnp.int32),
            pltpu.VMEM((2, CH), jnp.int32),
            pltpu.VMEM((16,), jnp.int32),
            pltpu.VMEM((CH, D), jnp.float32),
            pltpu.VMEM((CH, D), jnp.float32),
            pltpu.VMEM((CH, D), jnp.float32),
            pltpu.VMEM_SHARED((AR, D), jnp.float32),
            pltpu.SemaphoreType.DMA,
            pltpu.SemaphoreType.DMA,
        ],
    )
    def agg(g_hbm, plist_hbm, cnt_hbm, zrows_hbm, out0_hbm, out1_hbm,
            plist_v, sidx2, idx2, cnt_v, buf0, buf1, zbuf, acc, sem0, sem1):
        c = lax.axis_index("c")
        s = lax.axis_index("s")
        base = (c * NW + 2 * s) * LSZ
        pltpu.sync_copy(plist_hbm.at[pl.ds(base, 2 * LSZ)], plist_v)
        pltpu.sync_copy(cnt_hbm.at[pl.ds((c * NW + 2 * s) * 8, 16)], cnt_v)
        pltpu.sync_copy(zrows_hbm, zbuf)

        cv = cnt_v[...]
        nA = cv[0]
        nB = cv[8]
        n = nA + nB

        off = 0
        while off < RPT:
            m = min(CH, RPT - off)
            pltpu.sync_copy(zbuf.at[pl.ds(0, m)],
                            acc.at[pl.ds(s * RPT + off, m)])
            off += m
        plsc.subcore_barrier()

        def issue(j, pb, buf, sem):
            o = jnp.where(j < nA, j, CP + (j - nA)) * CH
            for l in range(CH // 16):
                w = plist_v[pl.ds(o + l * 16, 16)]
                sidx2[pb, pl.ds(l * 16, 16)] = lax.shift_right_logical(w, RB)
                idx2[pb, pl.ds(l * 16, 16)] = w & ((1 << RB) - 1)
            pltpu.async_copy(g_hbm.at[sidx2.at[pb]], buf, sem)

        @pl.when(n > 0)
        def _():
            issue(0, 0, buf0, sem0)

        @pl.when(n > 1)
        def _():
            issue(1, 1, buf1, sem1)

        def body(i, carry):
            j = 2 * i
            pltpu.make_async_copy(g_hbm.at[sidx2.at[0]], buf0, sem0).wait()
            pltpu.sync_copy(buf0, acc.at[idx2.at[0]], add=True)

            @pl.when(j + 2 < n)
            def _():
                issue(j + 2, 0, buf0, sem0)

            @pl.when(j + 1 < n)
            def _():
                pltpu.make_async_copy(g_hbm.at[sidx2.at[1]],
                                      buf1, sem1).wait()
                pltpu.sync_copy(buf1, acc.at[idx2.at[1]], add=True)

                @pl.when(j + 3 < n)
                def _():
                    issue(j + 3, 1, buf1, sem1)

            return carry

        lax.fori_loop(0, lax.shift_right_logical(n + 1, 1), body, 0)
        plsc.subcore_barrier()

        off = 0
        while off < RPT:
            m = min(CH, RPT - off)
            pltpu.sync_copy(acc.at[pl.ds(s * RPT + off, m)],
                            buf0.at[pl.ds(0, m)])

            @pl.when(c == 0)
            def _():
                pltpu.sync_copy(buf0.at[pl.ds(0, m)],
                                out0_hbm.at[pl.ds(s * RPT + off, m)])

            @pl.when(c == 1)
            def _():
                pltpu.sync_copy(buf0.at[pl.ds(0, m)],
                                out1_hbm.at[pl.ds(s * RPT + off, m)])

            off += m

    return agg


def _tc_first(d0, d1, x, W):
    N, D = x.shape

    def body(d0_ref, d1_ref, x_ref, w_ref, o_ref):
        dinv = lax.rsqrt(d0_ref[...] + d1_ref[...] + 1.0)
        h = jnp.dot(x_ref[...], w_ref[...], preferred_element_type=jnp.float32)
        o_ref[...] = dinv * h

    return pl.pallas_call(
        body,
        grid=(N // BN,),
        in_specs=[
            pl.BlockSpec((BN, 1), lambda i: (i, 0)),
            pl.BlockSpec((BN, 1), lambda i: (i, 0)),
            pl.BlockSpec((BN, D), lambda i: (i, 0)),
            pl.BlockSpec((D, D), lambda i: (0, 0)),
        ],
        out_specs=pl.BlockSpec((BN, D), lambda i: (i, 0)),
        out_shape=jax.ShapeDtypeStruct((N, D), jnp.float32),
    )(d0, d1, x, W)


def _part_specs(D, PB):
    return [
        pl.BlockSpec((BN, D), lambda i: (jnp.minimum(i, PB - 1), 0)),
        pl.BlockSpec((BN, D), lambda i: (jnp.maximum(i - PB, 0), 0)),
    ]


def _tc_mid(d0, d1, parts, g, b, W, PB):
    N, D = g.shape

    def body(d0_ref, d1_ref, p0, p1, g_ref, b_ref, w_ref, o_ref):
        i = pl.program_id(0)
        dinv = lax.rsqrt(d0_ref[...] + d1_ref[...] + 1.0)
        p = jnp.where(i < PB, p0[...], p1[...])
        xl = dinv * (p + g_ref[...]) + b_ref[...]
        xl = jnp.maximum(xl, 0.0)
        h = jnp.dot(xl, w_ref[...], preferred_element_type=jnp.float32)
        o_ref[...] = dinv * h

    return pl.pallas_call(
        body,
        grid=(N // BN,),
        in_specs=[
            pl.BlockSpec((BN, 1), lambda i: (i, 0)),
            pl.BlockSpec((BN, 1), lambda i: (i, 0)),
            *_part_specs(D, PB),
            pl.BlockSpec((BN, D), lambda i: (i, 0)),
            pl.BlockSpec((1, D), lambda i: (0, 0)),
            pl.BlockSpec((D, D), lambda i: (0, 0)),
        ],
        out_specs=pl.BlockSpec((BN, D), lambda i: (i, 0)),
        out_shape=jax.ShapeDtypeStruct((N, D), jnp.float32),
    )(d0, d1, *parts, g, b, W)


def _tc_last(d0, d1, parts, g, b, PB):
    N, D = g.shape

    def body(d0_ref, d1_ref, p0, p1, g_ref, b_ref, o_ref):
        i = pl.program_id(0)
        dinv = lax.rsqrt(d0_ref[...] + d1_ref[...] + 1.0)
        p = jnp.where(i < PB, p0[...], p1[...])
        o_ref[...] = dinv * (p + g_ref[...]) + b_ref[...]

    return pl.pallas_call(
        body,
        grid=(N // BN,),
        in_specs=[
            pl.BlockSpec((BN, 1), lambda i: (i, 0)),
            pl.BlockSpec((BN, 1), lambda i: (i, 0)),
            *_part_specs(D, PB),
            pl.BlockSpec((BN, D), lambda i: (i, 0)),
            pl.BlockSpec((1, D), lambda i: (0, 0)),
        ],
        out_specs=pl.BlockSpec((BN, D), lambda i: (i, 0)),
        out_shape=jax.ShapeDtypeStruct((N, D), jnp.float32),
    )(d0, d1, *parts, g, b)


def kernel(x, edge_index, W1, b1, W2, b2, W3, b3):
    N, D = x.shape
    E = edge_index.shape[1]
    assert N % (2 * BN) == 0 and D == 128

    C = -(-E // (NW * CH))
    C += C % 2
    E_pad = NW * C * CH
    CP = C + 1
    LSZ = CP * CH

    R1 = -(-(N + 1) // NS)
    R1 += (-R1) % 8
    D1 = R1 * NS

    H = N // 2
    PB = H // BN
    RPT = -(-(H + 2) // NS)
    RPT += (-RPT) % 8
    AR = RPT * NS
    TRASH = AR - 1
    assert N < (1 << (31 - RB)) and TRASH < (1 << RB)

    src = edge_index[0]
    dst = edge_index[1]
    pad = E_pad - E

    def _chunks(a):
        return a.reshape(C, NW, CH).transpose(1, 0, 2).reshape(NW * C, CH)

    srcs = _chunks(jnp.concatenate([src, jnp.zeros((pad,), src.dtype)]))
    dsts = _chunks(jnp.concatenate([dst, jnp.full((pad,), N, dst.dtype)]))

    zeros1 = jnp.zeros((R1,), jnp.float32)
    zrows = jnp.zeros((CH, D), jnp.float32)

    deg0, deg1 = _make_deg(C, D1, R1)(dsts, zeros1)
    d0 = deg0[:N].reshape(N, 1)
    d1 = deg1[:N].reshape(N, 1)

    plist, cnts = _make_part(C, CP, LSZ, H, TRASH)(srcs, dsts)

    agg = _make_agg(D, CP, LSZ, AR, RPT)
    b1r, b2r, b3r = (b.reshape(1, D) for b in (b1, b2, b3))

    g1 = _tc_first(d0, d1, x, W1)
    parts = agg(g1, plist, cnts, zrows)
    g2 = _tc_mid(d0, d1, parts, g1, b1r, W2, PB)
    parts = agg(g2, plist, cnts, zrows)
    g3 = _tc_mid(d0, d1, parts, g2, b2r, W3, PB)
    parts = agg(g3, plist, cnts, zrows)
    return _tc_last(d0, d1, parts, g3, b3r, PB)
